# baseline (speedup 1.0000x reference)
import functools

import jax
import jax.numpy as jnp
from jax import lax
from jax.experimental import pallas as pl
from jax.experimental.pallas import tpu as pltpu
from jax.experimental.pallas import tpu_sc as plsc

D_MODEL = 1024
HIDDEN = 2048
N_EXPERTS = 8
TOP_K = 2
N_TOKENS = 2048
CAP = 308
CAPP = 320
ROWS = N_EXPERTS * CAPP
ZROW = CAP
M_SLOTS = N_TOKENS * TOP_K
INT_MIN = -2147483648
ONE_F32_BITS = 0x3F800000

NW = 32
RPW = ROWS // NW
TPW = N_TOKENS // NW
HB = 512
NH = HIDDEN // HB


def _excl_cumsum_lanes(v):
    n = v.shape[1]
    liota = lax.broadcasted_iota(jnp.int32, v.shape, 1)
    acc = v
    sh = 1
    while sh < n:
        rolled = pltpu.roll(acc, sh, 1)
        acc = acc + jnp.where(liota >= sh, rolled, 0)
        sh *= 2
    return acc - v


def _router_body(x_ref, rwt_ref, rb_ref, comb_ref, gate_ref):
    x = x_ref[...]
    rwt = rwt_ref[...]
    logits = lax.dot_general(rwt, x, (((1,), (1,)), ((), ())),
                             preferred_element_type=jnp.float32)
    logits = logits + rb_ref[...]
    m = jnp.max(logits, axis=0, keepdims=True)
    z = jnp.exp(logits - m)
    p = z / jnp.sum(z, axis=0, keepdims=True)

    eiota = lax.broadcasted_iota(jnp.int32, p.shape, 0)
    g1 = jnp.max(p, axis=0, keepdims=True)
    id1 = jnp.min(jnp.where(p == g1, eiota, N_EXPERTS), axis=0, keepdims=True)
    k1 = eiota == id1
    p2 = jnp.where(k1, -1.0, p)
    g2 = jnp.max(p2, axis=0, keepdims=True)
    id2 = jnp.min(jnp.where(p2 == g2, eiota, N_EXPERTS), axis=0, keepdims=True)
    k2 = eiota == id2

    gi1 = lax.bitcast_convert_type(g1, jnp.int32)
    gi2 = lax.bitcast_convert_type(g2, jnp.int32)
    ia = jnp.where(k1, gi1, INT_MIN)
    ib = jnp.where(k2, gi2, INT_MIN)

    lo0 = jnp.full((N_EXPERTS, 1), -1, jnp.int32)
    hi0 = jnp.full((N_EXPERTS, 1), ONE_F32_BITS, jnp.int32)

    def bs_step(_, carry):
        lo, hi = carry
        mid = lo + (hi - lo + 1) // 2
        cnt = (jnp.sum(jnp.where(ia >= mid, 1, 0), axis=1, keepdims=True)
               + jnp.sum(jnp.where(ib >= mid, 1, 0), axis=1, keepdims=True))
        pred = cnt >= CAP
        return jnp.where(pred, mid, lo), jnp.where(pred, hi, mid - 1)

    thr, _ = lax.fori_loop(0, 32, bs_step, (lo0, hi0))

    n_above = (jnp.sum(jnp.where(ia > thr, 1, 0), axis=1, keepdims=True)
               + jnp.sum(jnp.where(ib > thr, 1, 0), axis=1, keepdims=True))
    ntie = CAP - n_above

    tie_a = jnp.where(ia == thr, 1, 0)
    tie_b = jnp.where(ib == thr, 1, 0)
    base_tie = _excl_cumsum_lanes(tie_a + tie_b)
    tr_a = base_tie
    tr_b = base_tie + tie_a
    keep_a = jnp.where(ia > thr, 1, jnp.where((tie_a > 0) & (tr_a < ntie), 1, 0))
    keep_b = jnp.where(ib > thr, 1, jnp.where((tie_b > 0) & (tr_b < ntie), 1, 0))

    base_pos = _excl_cumsum_lanes(keep_a + keep_b)
    pos_a = base_pos
    pos_b = base_pos + keep_a

    sel_pos_a = jnp.sum(jnp.where(keep_a > 0, pos_a, 0), axis=0, keepdims=True)
    sel_pos_b = jnp.sum(jnp.where(keep_b > 0, pos_b, 0), axis=0, keepdims=True)
    kept_a = jnp.sum(keep_a, axis=0, keepdims=True)
    kept_b = jnp.sum(keep_b, axis=0, keepdims=True)
    comb_a = jnp.where(kept_a > 0, id1 * CAPP + sel_pos_a, ZROW)
    comb_b = jnp.where(kept_b > 0, id2 * CAPP + sel_pos_b, ZROW)

    comb_ref[...] = jnp.concatenate([comb_a, comb_b], axis=0)
    gate_ref[...] = jnp.concatenate([g1, g2], axis=0)


def _router_call(x, rwt, rb_col):
    return pl.pallas_call(
        _router_body,
        out_shape=(
            jax.ShapeDtypeStruct((TOP_K, N_TOKENS), jnp.int32),
            jax.ShapeDtypeStruct((TOP_K, N_TOKENS), jnp.float32),
        ),
    )(x, rwt, rb_col)


@functools.cache
def _get_dispatch():
    mesh = plsc.VectorSubcoreMesh(core_axis_name="c", subcore_axis_name="s")

    @functools.partial(
        pl.kernel,
        mesh=mesh,
        compiler_params=pltpu.CompilerParams(needs_layout_passes=False),
        out_type=[
            jax.ShapeDtypeStruct((ROWS,), jnp.float32),
            jax.ShapeDtypeStruct((ROWS,), jnp.int32),
        ],
        scratch_types=[
            pltpu.VMEM((M_SLOTS,), jnp.int32),
            pltpu.VMEM((M_SLOTS,), jnp.float32),
            pltpu.VMEM((ROWS,), jnp.int32),
            pltpu.VMEM((ROWS,), jnp.float32),
        ],
    )
    def _dispatch(comb_hbm, gates_hbm, gpos_hbm, tpos_hbm,
                  comb_v, g_v, tok_v, gp_v):
        wid = lax.axis_index("c") * 16 + lax.axis_index("s")
        pltpu.sync_copy(comb_hbm, comb_v)
        pltpu.sync_copy(gates_hbm, g_v)

        zi = jnp.zeros((16,), jnp.int32)
        zf = jnp.zeros((16,), jnp.float32)

        def init_body(i, _):
            tok_v[pl.ds(i * 16, 16)] = zi
            gp_v[pl.ds(i * 16, 16)] = zf
            return 0

        lax.fori_loop(0, ROWS // 16, init_body, 0)

        vio = lax.broadcasted_iota(jnp.int32, (16,), 0)

        def scat_body(c, _):
            f0 = c * 16
            t0 = jnp.where(f0 >= N_TOKENS, f0 - N_TOKENS, f0)
            idx = comb_v[pl.ds(f0, 16)]
            gv = g_v[pl.ds(f0, 16)]
            msk = idx != ZROW
            plsc.store_scatter(tok_v, [idx], t0 + vio, mask=msk)
            plsc.store_scatter(gp_v, [idx], gv, mask=msk)
            return 0

        lax.fori_loop(0, M_SLOTS // 16, scat_body, 0)

        base = wid * RPW
        pltpu.sync_copy(gp_v.at[pl.ds(base, RPW)], gpos_hbm.at[pl.ds(base, RPW)])
        pltpu.sync_copy(tok_v.at[pl.ds(base, RPW)], tpos_hbm.at[pl.ds(base, RPW)])

    return _dispatch


HBF = 1024
NHB = HIDDEN // HBF


def _ffn_body(tpos_ref, x_ref, w1_ref, b1_ref, w2_ref, b2_ref, g_ref, out_ref,
              oh_ref, xe_ref):
    e = pl.program_id(0)
    hb = pl.program_id(1)
    g = g_ref[0]

    @pl.when(hb == 0)
    def _():
        tp = tpos_ref[0]
        tiota = lax.broadcasted_iota(jnp.int32, (N_TOKENS, CAPP), 0)
        oh_ref[...] = (tp == tiota).astype(jnp.bfloat16)
        xb = x_ref[...].astype(jnp.bfloat16)
        xe_ref[...] = lax.dot_general(
            oh_ref[...], xb, (((0,), (0,)), ((), ())),
            preferred_element_type=jnp.float32).astype(jnp.bfloat16)

    w1b = w1_ref[0].astype(jnp.bfloat16)
    z = jnp.dot(xe_ref[...], w1b,
                preferred_element_type=jnp.float32) + b1_ref[0]
    h = z / (1.0 + jnp.exp(-z))
    h = (h * g).astype(jnp.bfloat16)
    w2b = w2_ref[0].astype(jnp.bfloat16)
    yp = jnp.dot(h, w2b, preferred_element_type=jnp.float32)

    @pl.when(hb == 0)
    def _():
        yp2 = yp + g * b2_ref[0]
        ye = jnp.dot(oh_ref[...], yp2.astype(jnp.bfloat16),
                     preferred_element_type=jnp.float32)

        @pl.when(e == 0)
        def _():
            out_ref[...] = ye

        @pl.when(e != 0)
        def _():
            out_ref[...] = out_ref[...] + ye

    @pl.when(hb != 0)
    def _():
        ye = jnp.dot(oh_ref[...], yp.astype(jnp.bfloat16),
                     preferred_element_type=jnp.float32)
        out_ref[...] = out_ref[...] + ye


def _ffn_call(tpos3, x, w1, b1r, w2, b2r, gcol):
    return pl.pallas_call(
        _ffn_body,
        grid=(N_EXPERTS, NHB),
        in_specs=[
            pl.BlockSpec((1, 1, CAPP), lambda e, h: (e, 0, 0)),
            pl.BlockSpec((N_TOKENS, D_MODEL), lambda e, h: (0, 0)),
            pl.BlockSpec((1, D_MODEL, HBF), lambda e, h: (e, 0, h)),
            pl.BlockSpec((1, 1, HBF), lambda e, h: (e, 0, h)),
            pl.BlockSpec((1, HBF, D_MODEL), lambda e, h: (e, h, 0)),
            pl.BlockSpec((1, 1, D_MODEL), lambda e, h: (e, 0, 0)),
            pl.BlockSpec((1, CAPP, 1), lambda e, h: (e, 0, 0)),
        ],
        out_specs=pl.BlockSpec((N_TOKENS, D_MODEL), lambda e, h: (0, 0)),
        out_shape=jax.ShapeDtypeStruct((N_TOKENS, D_MODEL), jnp.float32),
        scratch_shapes=[
            pltpu.VMEM((N_TOKENS, CAPP), jnp.bfloat16),
            pltpu.VMEM((CAPP, D_MODEL), jnp.bfloat16),
        ],
    )(tpos3, x, w1, b1r, w2, b2r, gcol)


def kernel(x_btd, router_w, router_b, w1, b1, w2, b2):
    x = x_btd.reshape(N_TOKENS, D_MODEL)
    comb2, gates2 = _router_call(x, router_w.T, router_b.reshape(N_EXPERTS, 1))
    comb_flat = comb2.reshape(M_SLOTS)
    gates_flat = gates2.reshape(M_SLOTS)
    gpos, tpos = _get_dispatch()(comb_flat, gates_flat)
    gcol = gpos.reshape(N_EXPERTS, CAPP, 1)
    y = _ffn_call(tpos.reshape(N_EXPERTS, 1, CAPP), x,
                  w1, b1.reshape(N_EXPERTS, 1, HIDDEN),
                  w2, b2.reshape(N_EXPERTS, 1, D_MODEL), gcol)
    return y.reshape(1, N_TOKENS, D_MODEL)


# speedup vs baseline: 1.1585x; 1.1585x over previous
import functools

import jax
import jax.numpy as jnp
from jax import lax
from jax.experimental import pallas as pl
from jax.experimental.pallas import tpu as pltpu
from jax.experimental.pallas import tpu_sc as plsc

D_MODEL = 1024
HIDDEN = 2048
N_EXPERTS = 8
TOP_K = 2
N_TOKENS = 2048
CAP = 308
CAPP = 320
ROWS = N_EXPERTS * CAPP
ZROW = CAP
M_SLOTS = N_TOKENS * TOP_K
INT_MIN = -2147483648
ONE_F32_BITS = 0x3F800000

NW = 32
RPW = ROWS // NW
TPW = N_TOKENS // NW
HB = 512
NH = HIDDEN // HB


def _excl_cumsum_lanes(v):
    n = v.shape[1]
    liota = lax.broadcasted_iota(jnp.int32, v.shape, 1)
    acc = v
    sh = 1
    while sh < n:
        rolled = pltpu.roll(acc, sh, 1)
        acc = acc + jnp.where(liota >= sh, rolled, 0)
        sh *= 2
    return acc - v


def _router_body(x_ref, rwt_ref, rb_ref, comb_ref, gate_ref):
    x = x_ref[...]
    rwt = rwt_ref[...]
    logits = lax.dot_general(rwt, x, (((1,), (1,)), ((), ())),
                             preferred_element_type=jnp.float32)
    logits = logits + rb_ref[...]
    m = jnp.max(logits, axis=0, keepdims=True)
    z = jnp.exp(logits - m)
    p = z / jnp.sum(z, axis=0, keepdims=True)

    eiota = lax.broadcasted_iota(jnp.int32, p.shape, 0)
    g1 = jnp.max(p, axis=0, keepdims=True)
    id1 = jnp.min(jnp.where(p == g1, eiota, N_EXPERTS), axis=0, keepdims=True)
    k1 = eiota == id1
    p2 = jnp.where(k1, -1.0, p)
    g2 = jnp.max(p2, axis=0, keepdims=True)
    id2 = jnp.min(jnp.where(p2 == g2, eiota, N_EXPERTS), axis=0, keepdims=True)
    k2 = eiota == id2

    gi1 = lax.bitcast_convert_type(g1, jnp.int32)
    gi2 = lax.bitcast_convert_type(g2, jnp.int32)
    ia = jnp.where(k1, gi1, INT_MIN)
    ib = jnp.where(k2, gi2, INT_MIN)

    lo0 = jnp.full((N_EXPERTS, 1), -1, jnp.int32)
    hi0 = jnp.full((N_EXPERTS, 1), ONE_F32_BITS, jnp.int32)

    def bs_step(_, carry):
        lo, hi = carry
        mid = lo + (hi - lo + 1) // 2
        cnt = (jnp.sum(jnp.where(ia >= mid, 1, 0), axis=1, keepdims=True)
               + jnp.sum(jnp.where(ib >= mid, 1, 0), axis=1, keepdims=True))
        pred = cnt >= CAP
        return jnp.where(pred, mid, lo), jnp.where(pred, hi, mid - 1)

    thr, _ = lax.fori_loop(0, 32, bs_step, (lo0, hi0))

    n_above = (jnp.sum(jnp.where(ia > thr, 1, 0), axis=1, keepdims=True)
               + jnp.sum(jnp.where(ib > thr, 1, 0), axis=1, keepdims=True))
    ntie = CAP - n_above

    tie_a = jnp.where(ia == thr, 1, 0)
    tie_b = jnp.where(ib == thr, 1, 0)
    base_tie = _excl_cumsum_lanes(tie_a + tie_b)
    tr_a = base_tie
    tr_b = base_tie + tie_a
    keep_a = jnp.where(ia > thr, 1, jnp.where((tie_a > 0) & (tr_a < ntie), 1, 0))
    keep_b = jnp.where(ib > thr, 1, jnp.where((tie_b > 0) & (tr_b < ntie), 1, 0))

    base_pos = _excl_cumsum_lanes(keep_a + keep_b)
    pos_a = base_pos
    pos_b = base_pos + keep_a

    sel_pos_a = jnp.sum(jnp.where(keep_a > 0, pos_a, 0), axis=0, keepdims=True)
    sel_pos_b = jnp.sum(jnp.where(keep_b > 0, pos_b, 0), axis=0, keepdims=True)
    kept_a = jnp.sum(keep_a, axis=0, keepdims=True)
    kept_b = jnp.sum(keep_b, axis=0, keepdims=True)
    comb_a = jnp.where(kept_a > 0, id1 * CAPP + sel_pos_a, ZROW)
    comb_b = jnp.where(kept_b > 0, id2 * CAPP + sel_pos_b, ZROW)

    comb_ref[...] = jnp.concatenate([comb_a, comb_b], axis=0)
    gate_ref[...] = jnp.concatenate([g1, g2], axis=0)


def _router_call(x, rwt, rb_col):
    return pl.pallas_call(
        _router_body,
        out_shape=(
            jax.ShapeDtypeStruct((TOP_K, N_TOKENS), jnp.int32),
            jax.ShapeDtypeStruct((TOP_K, N_TOKENS), jnp.float32),
        ),
    )(x, rwt, rb_col)


@functools.cache
def _get_dispatch():
    mesh = plsc.VectorSubcoreMesh(core_axis_name="c", subcore_axis_name="s")

    @functools.partial(
        pl.kernel,
        mesh=mesh,
        compiler_params=pltpu.CompilerParams(needs_layout_passes=False),
        out_type=[
            jax.ShapeDtypeStruct((ROWS,), jnp.float32),
            jax.ShapeDtypeStruct((ROWS,), jnp.int32),
        ],
        scratch_types=[
            pltpu.VMEM((M_SLOTS,), jnp.int32),
            pltpu.VMEM((M_SLOTS,), jnp.float32),
            pltpu.VMEM((ROWS,), jnp.int32),
            pltpu.VMEM((ROWS,), jnp.float32),
        ],
    )
    def _dispatch(comb_hbm, gates_hbm, gpos_hbm, tpos_hbm,
                  comb_v, g_v, tok_v, gp_v):
        wid = lax.axis_index("c") * 16 + lax.axis_index("s")
        pltpu.sync_copy(comb_hbm, comb_v)
        pltpu.sync_copy(gates_hbm, g_v)

        zi = jnp.zeros((16,), jnp.int32)
        zf = jnp.zeros((16,), jnp.float32)

        def init_body(i, _):
            tok_v[pl.ds(i * 16, 16)] = zi
            gp_v[pl.ds(i * 16, 16)] = zf
            return 0

        lax.fori_loop(0, ROWS // 16, init_body, 0)

        vio = lax.broadcasted_iota(jnp.int32, (16,), 0)

        def scat_body(c, _):
            f0 = c * 16
            t0 = jnp.where(f0 >= N_TOKENS, f0 - N_TOKENS, f0)
            idx = comb_v[pl.ds(f0, 16)]
            gv = g_v[pl.ds(f0, 16)]
            msk = idx != ZROW
            plsc.store_scatter(tok_v, [idx], t0 + vio, mask=msk)
            plsc.store_scatter(gp_v, [idx], gv, mask=msk)
            return 0

        lax.fori_loop(0, M_SLOTS // 16, scat_body, 0)

        base = wid * RPW
        pltpu.sync_copy(gp_v.at[pl.ds(base, RPW)], gpos_hbm.at[pl.ds(base, RPW)])
        pltpu.sync_copy(tok_v.at[pl.ds(base, RPW)], tpos_hbm.at[pl.ds(base, RPW)])

    return _dispatch


def _ffn_body(tpos_ref, x_ref, w1_ref, b1_ref, w2_ref, b2_ref, g_ref, out_ref):
    e = pl.program_id(0)
    tp = tpos_ref[0]
    tiota = lax.broadcasted_iota(jnp.int32, (N_TOKENS, CAPP), 0)
    oh = (tp == tiota).astype(jnp.bfloat16)
    x = lax.dot_general(oh, x_ref[...], (((0,), (0,)), ((), ())),
                        preferred_element_type=jnp.float32
                        ).astype(jnp.bfloat16)
    w1b = w1_ref[0].astype(jnp.bfloat16)
    z = jnp.dot(x, w1b, preferred_element_type=jnp.float32) + b1_ref[0]
    h = z / (1.0 + jnp.exp(-z))
    g = g_ref[0]
    h = (h * g).astype(jnp.bfloat16)
    w2b = w2_ref[0].astype(jnp.bfloat16)
    yp = jnp.dot(h, w2b, preferred_element_type=jnp.float32)
    yde = (g * b2_ref[0] + yp).astype(jnp.bfloat16)
    ye = jnp.dot(oh, yde, preferred_element_type=jnp.float32)

    @pl.when(e == 0)
    def _():
        out_ref[...] = ye

    @pl.when(e != 0)
    def _():
        out_ref[...] = out_ref[...] + ye


def _ffn_call(tpos3, xb16, w1, b1r, w2, b2r, gcol):
    return pl.pallas_call(
        _ffn_body,
        grid=(N_EXPERTS,),
        in_specs=[
            pl.BlockSpec((1, 1, CAPP), lambda e: (e, 0, 0)),
            pl.BlockSpec((N_TOKENS, D_MODEL), lambda e: (0, 0)),
            pl.BlockSpec((1, D_MODEL, HIDDEN), lambda e: (e, 0, 0)),
            pl.BlockSpec((1, 1, HIDDEN), lambda e: (e, 0, 0)),
            pl.BlockSpec((1, HIDDEN, D_MODEL), lambda e: (e, 0, 0)),
            pl.BlockSpec((1, 1, D_MODEL), lambda e: (e, 0, 0)),
            pl.BlockSpec((1, CAPP, 1), lambda e: (e, 0, 0)),
        ],
        out_specs=pl.BlockSpec((N_TOKENS, D_MODEL), lambda e: (0, 0)),
        out_shape=jax.ShapeDtypeStruct((N_TOKENS, D_MODEL), jnp.float32),
    )(tpos3, xb16, w1, b1r, w2, b2r, gcol)


def kernel(x_btd, router_w, router_b, w1, b1, w2, b2):
    x = x_btd.reshape(N_TOKENS, D_MODEL)
    comb2, gates2 = _router_call(x, router_w.T, router_b.reshape(N_EXPERTS, 1))
    comb_flat = comb2.reshape(M_SLOTS)
    gates_flat = gates2.reshape(M_SLOTS)
    gpos, tpos = _get_dispatch()(comb_flat, gates_flat)
    gcol = gpos.reshape(N_EXPERTS, CAPP, 1)
    y = _ffn_call(tpos.reshape(N_EXPERTS, 1, CAPP), x.astype(jnp.bfloat16),
                  w1, b1.reshape(N_EXPERTS, 1, HIDDEN),
                  w2, b2.reshape(N_EXPERTS, 1, D_MODEL), gcol)
    return y.reshape(1, N_TOKENS, D_MODEL)


# speedup vs baseline: 1.1735x; 1.0130x over previous
import functools

import jax
import jax.numpy as jnp
from jax import lax
from jax.experimental import pallas as pl
from jax.experimental.pallas import tpu as pltpu
from jax.experimental.pallas import tpu_sc as plsc

D_MODEL = 1024
HIDDEN = 2048
N_EXPERTS = 8
TOP_K = 2
N_TOKENS = 2048
CAP = 308
CAPP = 320
ROWS = N_EXPERTS * CAPP
ZROW = CAP
M_SLOTS = N_TOKENS * TOP_K
INT_MIN = -2147483648
ONE_F32_BITS = 0x3F800000

NW = 32
RPW = ROWS // NW
TPW = N_TOKENS // NW
HB = 512
NH = HIDDEN // HB


def _excl_cumsum_lanes(v):
    n = v.shape[1]
    liota = lax.broadcasted_iota(jnp.int32, v.shape, 1)
    acc = v
    sh = 1
    while sh < n:
        rolled = pltpu.roll(acc, sh, 1)
        acc = acc + jnp.where(liota >= sh, rolled, 0)
        sh *= 2
    return acc - v


def _router_body(x_ref, rwt_ref, rb_ref, comb_ref, gate_ref):
    x = x_ref[...]
    rwt = rwt_ref[...]
    logits = lax.dot_general(rwt, x, (((1,), (1,)), ((), ())),
                             preferred_element_type=jnp.float32)
    logits = logits + rb_ref[...]
    m = jnp.max(logits, axis=0, keepdims=True)
    z = jnp.exp(logits - m)
    p = z / jnp.sum(z, axis=0, keepdims=True)

    eiota = lax.broadcasted_iota(jnp.int32, p.shape, 0)
    g1 = jnp.max(p, axis=0, keepdims=True)
    id1 = jnp.min(jnp.where(p == g1, eiota, N_EXPERTS), axis=0, keepdims=True)
    k1 = eiota == id1
    p2 = jnp.where(k1, -1.0, p)
    g2 = jnp.max(p2, axis=0, keepdims=True)
    id2 = jnp.min(jnp.where(p2 == g2, eiota, N_EXPERTS), axis=0, keepdims=True)
    k2 = eiota == id2

    gi1 = lax.bitcast_convert_type(g1, jnp.int32)
    gi2 = lax.bitcast_convert_type(g2, jnp.int32)
    ia = jnp.where(k1, gi1, INT_MIN)
    ib = jnp.where(k2, gi2, INT_MIN)

    lo0 = jnp.full((N_EXPERTS, 1), -1, jnp.int32)
    hi0 = jnp.full((N_EXPERTS, 1), ONE_F32_BITS, jnp.int32)

    def bs_step(_, carry):
        lo, hi = carry
        mid = lo + (hi - lo + 1) // 2
        cnt = (jnp.sum(jnp.where(ia >= mid, 1, 0), axis=1, keepdims=True)
               + jnp.sum(jnp.where(ib >= mid, 1, 0), axis=1, keepdims=True))
        pred = cnt >= CAP
        return jnp.where(pred, mid, lo), jnp.where(pred, hi, mid - 1)

    thr, _ = lax.fori_loop(0, 32, bs_step, (lo0, hi0))

    n_above = (jnp.sum(jnp.where(ia > thr, 1, 0), axis=1, keepdims=True)
               + jnp.sum(jnp.where(ib > thr, 1, 0), axis=1, keepdims=True))
    ntie = CAP - n_above

    tie_a = jnp.where(ia == thr, 1, 0)
    tie_b = jnp.where(ib == thr, 1, 0)
    base_tie = _excl_cumsum_lanes(tie_a + tie_b)
    tr_a = base_tie
    tr_b = base_tie + tie_a
    keep_a = jnp.where(ia > thr, 1, jnp.where((tie_a > 0) & (tr_a < ntie), 1, 0))
    keep_b = jnp.where(ib > thr, 1, jnp.where((tie_b > 0) & (tr_b < ntie), 1, 0))

    base_pos = _excl_cumsum_lanes(keep_a + keep_b)
    pos_a = base_pos
    pos_b = base_pos + keep_a

    sel_pos_a = jnp.sum(jnp.where(keep_a > 0, pos_a, 0), axis=0, keepdims=True)
    sel_pos_b = jnp.sum(jnp.where(keep_b > 0, pos_b, 0), axis=0, keepdims=True)
    kept_a = jnp.sum(keep_a, axis=0, keepdims=True)
    kept_b = jnp.sum(keep_b, axis=0, keepdims=True)
    comb_a = jnp.where(kept_a > 0, id1 * CAPP + sel_pos_a, ZROW)
    comb_b = jnp.where(kept_b > 0, id2 * CAPP + sel_pos_b, ZROW)

    comb_ref[...] = jnp.concatenate([comb_a, comb_b], axis=0)
    gate_ref[...] = jnp.concatenate([g1, g2], axis=0)


def _router_call(x, rwt, rb_col):
    return pl.pallas_call(
        _router_body,
        out_shape=(
            jax.ShapeDtypeStruct((TOP_K, N_TOKENS), jnp.int32),
            jax.ShapeDtypeStruct((TOP_K, N_TOKENS), jnp.float32),
        ),
    )(x, rwt, rb_col)


@functools.cache
def _get_dispatch():
    mesh = plsc.VectorSubcoreMesh(core_axis_name="c", subcore_axis_name="s")

    @functools.partial(
        pl.kernel,
        mesh=mesh,
        compiler_params=pltpu.CompilerParams(needs_layout_passes=False),
        out_type=[
            jax.ShapeDtypeStruct((ROWS,), jnp.float32),
            jax.ShapeDtypeStruct((ROWS,), jnp.int32),
        ],
        scratch_types=[
            pltpu.VMEM((M_SLOTS,), jnp.int32),
            pltpu.VMEM((M_SLOTS,), jnp.float32),
            pltpu.VMEM((ROWS,), jnp.int32),
            pltpu.VMEM((ROWS,), jnp.float32),
        ],
    )
    def _dispatch(comb_hbm, gates_hbm, gpos_hbm, tpos_hbm,
                  comb_v, g_v, tok_v, gp_v):
        wid = lax.axis_index("c") * 16 + lax.axis_index("s")
        pltpu.sync_copy(comb_hbm, comb_v)
        pltpu.sync_copy(gates_hbm, g_v)

        zi = jnp.zeros((16,), jnp.int32)
        zf = jnp.zeros((16,), jnp.float32)

        def init_body(i, _):
            tok_v[pl.ds(i * 16, 16)] = zi
            gp_v[pl.ds(i * 16, 16)] = zf
            return 0

        lax.fori_loop(0, ROWS // 16, init_body, 0)

        vio = lax.broadcasted_iota(jnp.int32, (16,), 0)

        def scat_body(c, _):
            f0 = c * 16
            t0 = jnp.where(f0 >= N_TOKENS, f0 - N_TOKENS, f0)
            idx = comb_v[pl.ds(f0, 16)]
            gv = g_v[pl.ds(f0, 16)]
            msk = idx != ZROW
            plsc.store_scatter(tok_v, [idx], t0 + vio, mask=msk)
            plsc.store_scatter(gp_v, [idx], gv, mask=msk)
            return 0

        lax.fori_loop(0, M_SLOTS // 16, scat_body, 0)

        base = wid * RPW
        pltpu.sync_copy(gp_v.at[pl.ds(base, RPW)], gpos_hbm.at[pl.ds(base, RPW)])
        pltpu.sync_copy(tok_v.at[pl.ds(base, RPW)], tpos_hbm.at[pl.ds(base, RPW)])

    return _dispatch


def _ffn_body(tpos_ref, x_ref, w1_ref, b1_ref, w2_ref, b2_ref, g_ref, out_ref):
    e = pl.program_id(0)
    tp = tpos_ref[0]
    tiota = lax.broadcasted_iota(jnp.int32, (N_TOKENS, CAPP), 0)
    oh = (tp == tiota).astype(jnp.bfloat16)
    xb = x_ref[...].astype(jnp.bfloat16)
    x = lax.dot_general(oh, xb, (((0,), (0,)), ((), ())),
                        preferred_element_type=jnp.float32
                        ).astype(jnp.bfloat16)
    w1b = w1_ref[0].astype(jnp.bfloat16)
    z = jnp.dot(x, w1b, preferred_element_type=jnp.float32) + b1_ref[0]
    h = z / (1.0 + jnp.exp(-z))
    g = g_ref[0]
    h = (h * g).astype(jnp.bfloat16)
    w2b = w2_ref[0].astype(jnp.bfloat16)
    yp = jnp.dot(h, w2b, preferred_element_type=jnp.float32)
    yde = (g * b2_ref[0] + yp).astype(jnp.bfloat16)
    ye = jnp.dot(oh, yde, preferred_element_type=jnp.float32)

    @pl.when(e == 0)
    def _():
        out_ref[...] = ye

    @pl.when(e != 0)
    def _():
        out_ref[...] = out_ref[...] + ye


def _ffn_call(tpos3, xb16, w1, b1r, w2, b2r, gcol):
    return pl.pallas_call(
        _ffn_body,
        grid=(N_EXPERTS,),
        in_specs=[
            pl.BlockSpec((1, 1, CAPP), lambda e: (e, 0, 0)),
            pl.BlockSpec((N_TOKENS, D_MODEL), lambda e: (0, 0)),
            pl.BlockSpec((1, D_MODEL, HIDDEN), lambda e: (e, 0, 0)),
            pl.BlockSpec((1, 1, HIDDEN), lambda e: (e, 0, 0)),
            pl.BlockSpec((1, HIDDEN, D_MODEL), lambda e: (e, 0, 0)),
            pl.BlockSpec((1, 1, D_MODEL), lambda e: (e, 0, 0)),
            pl.BlockSpec((1, CAPP, 1), lambda e: (e, 0, 0)),
        ],
        out_specs=pl.BlockSpec((N_TOKENS, D_MODEL), lambda e: (0, 0)),
        out_shape=jax.ShapeDtypeStruct((N_TOKENS, D_MODEL), jnp.float32),
    )(tpos3, xb16, w1, b1r, w2, b2r, gcol)


def kernel(x_btd, router_w, router_b, w1, b1, w2, b2):
    x = x_btd.reshape(N_TOKENS, D_MODEL)
    comb2, gates2 = _router_call(x, router_w.T, router_b.reshape(N_EXPERTS, 1))
    comb_flat = comb2.reshape(M_SLOTS)
    gates_flat = gates2.reshape(M_SLOTS)
    gpos, tpos = _get_dispatch()(comb_flat, gates_flat)
    gcol = gpos.reshape(N_EXPERTS, CAPP, 1)
    y = _ffn_call(tpos.reshape(N_EXPERTS, 1, CAPP), x,
                  w1, b1.reshape(N_EXPERTS, 1, HIDDEN),
                  w2, b2.reshape(N_EXPERTS, 1, D_MODEL), gcol)
    return y.reshape(1, N_TOKENS, D_MODEL)
